# baseline (speedup 1.0000x reference)
import functools

import jax
import jax.numpy as jnp
from jax import lax
from jax.experimental import pallas as pl
from jax.experimental.pallas import tpu as pltpu
from jax.experimental.pallas import tpu_sc as plsc

NC = 2
NS = 16
NW = NC * NS

GW = 128
NBUF = 5
KAHEAD = 3


def _body(rows_per_w, n_chunks, D,
          table_hbm, idx_hbm, out_hbm, idx_v, rows_v, gsem, osem):
    wid = lax.axis_index("s") * NC + lax.axis_index("c")
    base = wid * rows_per_w

    pltpu.sync_copy(idx_hbm.at[pl.ds(base, rows_per_w)], idx_v)

    def fire_gather(buf, c):
        @pl.loop(0, GW // 16, unroll=2)
        def _grp(g):
            v = idx_v[c, pl.ds(g * 16, 16)]
            for l in range(16):
                r = v[l]
                pltpu.async_copy(
                    table_hbm.at[pl.ds(r // 8, 1), pl.ds(r % 8, 1)],
                    rows_v.at[buf].at[pl.ds(g * 16 + l, 1)].reshape(1, 1, 64),
                    gsem)

    def drain_gather(buf, c):
        pltpu.make_async_copy(
            table_hbm.at[pl.ds(0, GW // 8)], rows_v.at[buf].reshape(GW // 8, 8, D),
            gsem).wait()

    def fire_write(buf, c):
        pltpu.async_copy(rows_v.at[buf], out_hbm.at[base + c], osem)

    def wait_write(buf, c):
        pltpu.make_async_copy(
            rows_v.at[buf], out_hbm.at[base + c], osem).wait()

    for p in range(KAHEAD):
        fire_gather(p, p)

    @pl.loop(0, n_chunks, step=NBUF)
    def _super(s):
        for b in range(NBUF):
            c = s + b
            fbuf = (b + KAHEAD) % NBUF
            if b < NBUF - KAHEAD:
                @pl.when(s > 0)
                def _():
                    wait_write(fbuf, c - (NBUF - KAHEAD))
            else:
                wait_write(fbuf, c - (NBUF - KAHEAD))
            if b < NBUF - KAHEAD:
                fire_gather(fbuf, c + KAHEAD)
            else:
                @pl.when(s < n_chunks - NBUF)
                def _():
                    fire_gather(fbuf, c + KAHEAD)
            drain_gather(b, c)
            fire_write(b, c)

    wait_write((n_chunks - 2) % NBUF, n_chunks - 2)
    wait_write((n_chunks - 1) % NBUF, n_chunks - 1)


def kernel(x, embedding):
    B0, B1 = x.shape
    V, D = embedding.shape
    B = B0 * B1
    rows_total = B // GW
    rows_per_w = rows_total // NW
    n_chunks = rows_per_w

    idx = x.reshape(rows_total, GW).astype(jnp.int32)

    mesh = plsc.VectorSubcoreMesh(core_axis_name="c", subcore_axis_name="s")
    body = functools.partial(_body, rows_per_w, n_chunks, D)
    out = pl.kernel(
        body,
        out_type=jax.ShapeDtypeStruct((rows_total, GW, D), jnp.float32),
        mesh=mesh,
        scratch_types=[
            pltpu.VMEM((rows_per_w, GW), jnp.int32),
            pltpu.VMEM((NBUF, GW, D), jnp.float32),
            pltpu.SemaphoreType.DMA,
            pltpu.SemaphoreType.DMA,
        ],
        compiler_params=pltpu.CompilerParams(use_tc_tiling_on_sc=True),
    )(embedding.reshape(V // 8, 8, D), idx)
    return out.reshape(B0, B1, D)


# speedup vs baseline: 1.0202x; 1.0202x over previous
import functools

import jax
import jax.numpy as jnp
from jax import lax
from jax.experimental import pallas as pl
from jax.experimental.pallas import tpu as pltpu
from jax.experimental.pallas import tpu_sc as plsc

NC = 2
NS = 16
NW = NC * NS

GW = 128
NBUF = 5
KAHEAD = 3


def _body(rows_per_w, n_chunks, D,
          table_hbm, idx_hbm, out_hbm, idx_v, rows_v, gsem, osem):
    wid = lax.axis_index("s") * NC + lax.axis_index("c")
    base = wid * rows_per_w

    pltpu.sync_copy(idx_hbm.at[pl.ds(base, rows_per_w)], idx_v)

    def fire_gather(buf, c):
        @pl.loop(0, GW // 16)
        def _grp(g):
            v = idx_v[c, pl.ds(g * 16, 16)]
            for l in range(16):
                r = v[l]
                pltpu.async_copy(
                    table_hbm.at[pl.ds(r // 8, 1), pl.ds(r % 8, 1)],
                    rows_v.at[buf].at[pl.ds(g * 16 + l, 1)].reshape(1, 1, 64),
                    gsem)

    def drain_gather(buf, c):
        pltpu.make_async_copy(
            table_hbm.at[pl.ds(0, GW // 8)], rows_v.at[buf].reshape(GW // 8, 8, D),
            gsem).wait()

    def fire_write(buf, c):
        pltpu.async_copy(rows_v.at[buf], out_hbm.at[base + c], osem)

    def wait_write(buf, c):
        pltpu.make_async_copy(
            rows_v.at[buf], out_hbm.at[base + c], osem).wait()

    for p in range(KAHEAD):
        fire_gather(p, p)

    @pl.loop(0, n_chunks, step=NBUF)
    def _super(s):
        for b in range(NBUF):
            c = s + b
            fbuf = (b + KAHEAD) % NBUF
            if b < NBUF - KAHEAD:
                @pl.when(s > 0)
                def _():
                    wait_write(fbuf, c - (NBUF - KAHEAD))
            else:
                wait_write(fbuf, c - (NBUF - KAHEAD))
            if b < NBUF - KAHEAD:
                fire_gather(fbuf, c + KAHEAD)
            else:
                @pl.when(s < n_chunks - NBUF)
                def _():
                    fire_gather(fbuf, c + KAHEAD)
            drain_gather(b, c)
            fire_write(b, c)

    wait_write((n_chunks - 2) % NBUF, n_chunks - 2)
    wait_write((n_chunks - 1) % NBUF, n_chunks - 1)


def kernel(x, embedding):
    B0, B1 = x.shape
    V, D = embedding.shape
    B = B0 * B1
    rows_total = B // GW
    rows_per_w = rows_total // NW
    n_chunks = rows_per_w

    idx = x.reshape(rows_total, GW).astype(jnp.int32)

    mesh = plsc.VectorSubcoreMesh(core_axis_name="c", subcore_axis_name="s")
    body = functools.partial(_body, rows_per_w, n_chunks, D)
    out = pl.kernel(
        body,
        out_type=jax.ShapeDtypeStruct((rows_total, GW, D), jnp.float32),
        mesh=mesh,
        scratch_types=[
            pltpu.VMEM((rows_per_w, GW), jnp.int32),
            pltpu.VMEM((NBUF, GW, D), jnp.float32),
            pltpu.SemaphoreType.DMA,
            pltpu.SemaphoreType.DMA,
        ],
        compiler_params=pltpu.CompilerParams(use_tc_tiling_on_sc=True),
    )(embedding.reshape(V // 8, 8, D), idx)
    return out.reshape(B0, B1, D)
